# Initial kernel scaffold; baseline (speedup 1.0000x reference)
import jax
import jax.numpy as jnp
from jax.experimental import pallas as pl


def kernel(seq, time_stamp, time_embed):
    raise NotImplementedError("write your pallas kernel here")


# speedup vs baseline: 2.7470x; 2.7470x over previous
import jax
import jax.numpy as jnp
from jax.experimental import pallas as pl

_NROWS = 49
_CHUNK = 512


def _body(idx_ref, seq_ref, tab_ref, out_ref):
    ids = idx_ref[0, 0, :]
    ids = jnp.where(ids == -1, _NROWS - 1, ids)
    onehot = (
        ids[:, None] == jax.lax.broadcasted_iota(jnp.int32, (1, _NROWS), 1)
    ).astype(jnp.float32)
    emb = jax.lax.dot_general(
        onehot,
        tab_ref[...],
        (((1,), (0,)), ((), ())),
        preferred_element_type=jnp.float32,
    )
    out_ref[...] = seq_ref[...] + emb


def kernel(seq, time_stamp, time_embed):
    B, L, D = seq.shape
    n = B * L
    seq2 = seq.reshape(n, D)
    idx3 = time_stamp.reshape(-1).astype(jnp.int32).reshape(n // _CHUNK, 1, _CHUNK)
    out = pl.pallas_call(
        _body,
        grid=(n // _CHUNK,),
        in_specs=[
            pl.BlockSpec((1, 1, _CHUNK), lambda i: (i, 0, 0)),
            pl.BlockSpec((_CHUNK, D), lambda i: (i, 0)),
            pl.BlockSpec((_NROWS, D), lambda i: (0, 0)),
        ],
        out_specs=pl.BlockSpec((_CHUNK, D), lambda i: (i, 0)),
        out_shape=jax.ShapeDtypeStruct((n, D), jnp.float32),
    )(idx3, seq2, time_embed)
    return out.reshape(B, L, D)


# speedup vs baseline: 3.9875x; 1.4516x over previous
import functools

import jax
import jax.numpy as jnp
from jax import lax
from jax.experimental import pallas as pl
from jax.experimental.pallas import tpu as pltpu
from jax.experimental.pallas import tpu_sc as plsc

_N_TAB = 49
_D = 128
_C = 128
_NC = 2
_NS = 16
_NW = _NC * _NS


def _sc_body(nsteps, seq_hbm, idx_hbm, tab_hbm, out_hbm,
             tab_sh, idx_v, iota_v, emb_v, seq_v, gsem):
    cid = lax.axis_index("c")
    sid = lax.axis_index("s")
    wid = sid * _NC + cid
    rows_per_w = nsteps * _C

    @pl.when(sid == 0)
    def _():
        pltpu.sync_copy(tab_hbm, tab_sh)

    for i in range(_C // 16):
        iota_v[pl.ds(i * 16, 16)] = lax.iota(jnp.int32, 16) + (i * 16)

    plsc.subcore_barrier()

    def step(g, carry):
        base = wid * rows_per_w + g * _C
        pltpu.sync_copy(idx_hbm.at[pl.ds(base, _C)], idx_v)
        for i in range(_C // 16):
            v = idx_v[pl.ds(i * 16, 16)]
            idx_v[pl.ds(i * 16, 16)] = jnp.where(v == -1, _N_TAB - 1, v)
        gather = pltpu.async_copy(tab_sh.at[idx_v], emb_v, gsem)
        pltpu.sync_copy(seq_hbm.at[pl.ds(base, _C)], seq_v)
        gather.wait()

        def add_row(r, c):
            for d in range(_D // 16):
                sl = pl.ds(d * 16, 16)
                seq_v[r, sl] = seq_v[r, sl] + emb_v[r, sl]
            return c

        lax.fori_loop(0, _C, add_row, 0)
        pltpu.sync_copy(seq_v, out_hbm.at[pl.ds(base, _C)])
        return carry

    lax.fori_loop(0, nsteps, step, 0)


def kernel(seq, time_stamp, time_embed):
    B, L, D = seq.shape
    n = B * L
    seq2 = seq.reshape(n, D)
    idx = time_stamp.reshape(-1).astype(jnp.int32)
    nsteps = n // (_NW * _C)
    mesh = plsc.VectorSubcoreMesh(core_axis_name="c", subcore_axis_name="s")
    out = pl.kernel(
        functools.partial(_sc_body, nsteps),
        out_type=jax.ShapeDtypeStruct((n, D), jnp.float32),
        mesh=mesh,
        scratch_types=[
            pltpu.VMEM_SHARED((_N_TAB, _D), jnp.float32),
            pltpu.VMEM((_C,), jnp.int32),
            pltpu.VMEM((_C,), jnp.int32),
            pltpu.VMEM((_C, _D), jnp.float32),
            pltpu.VMEM((_C, _D), jnp.float32),
            pltpu.SemaphoreType.DMA,
        ],
    )(seq2, idx, time_embed)
    return out.reshape(B, L, D)


# speedup vs baseline: 4.5539x; 1.1421x over previous
import functools

import jax
import jax.numpy as jnp
from jax import lax
from jax.experimental import pallas as pl
from jax.experimental.pallas import tpu as pltpu
from jax.experimental.pallas import tpu_sc as plsc

_N_TAB = 49
_D = 128
_C = 128
_NC = 2
_NS = 16
_NW = _NC * _NS


def _sc_body(nsteps, seq_hbm, idx_hbm, tab_hbm, out_hbm,
             tab_sh, idx_v, iota_v, emb_v, seq_v, gsem):
    cid = lax.axis_index("c")
    sid = lax.axis_index("s")
    wid = sid * _NC + cid
    rows_per_w = nsteps * _C

    @pl.when(sid == 0)
    def _():
        pltpu.sync_copy(tab_hbm, tab_sh)

    for i in range(_C // 16):
        iota_v[pl.ds(i * 16, 16)] = lax.iota(jnp.int32, 16) + (i * 16)

    plsc.subcore_barrier()

    def step(g, carry):
        base = wid * rows_per_w + g * _C
        pltpu.sync_copy(idx_hbm.at[pl.ds(base, _C)], idx_v)
        for i in range(_C // 16):
            v = idx_v[pl.ds(i * 16, 16)]
            idx_v[pl.ds(i * 16, 16)] = jnp.where(v == -1, _N_TAB - 1, v)
        pltpu.sync_copy(seq_hbm.at[pl.ds(base, _C)], seq_v)
        pltpu.sync_copy(tab_sh.at[idx_v], seq_v, add=True)
        pltpu.sync_copy(seq_v, out_hbm.at[pl.ds(base, _C)])
        return carry

    lax.fori_loop(0, nsteps, step, 0)


def kernel(seq, time_stamp, time_embed):
    B, L, D = seq.shape
    n = B * L
    seq2 = seq.reshape(n, D)
    idx = time_stamp.reshape(-1).astype(jnp.int32)
    nsteps = n // (_NW * _C)
    mesh = plsc.VectorSubcoreMesh(core_axis_name="c", subcore_axis_name="s")
    out = pl.kernel(
        functools.partial(_sc_body, nsteps),
        out_type=jax.ShapeDtypeStruct((n, D), jnp.float32),
        mesh=mesh,
        scratch_types=[
            pltpu.VMEM_SHARED((_N_TAB, _D), jnp.float32),
            pltpu.VMEM((_C,), jnp.int32),
            pltpu.VMEM((_C,), jnp.int32),
            pltpu.VMEM((_C, _D), jnp.float32),
            pltpu.VMEM((_C, _D), jnp.float32),
            pltpu.SemaphoreType.DMA,
        ],
    )(seq2, idx, time_embed)
    return out.reshape(B, L, D)


# speedup vs baseline: 9.4169x; 2.0679x over previous
import functools

import jax
import jax.numpy as jnp
from jax import lax
from jax.experimental import pallas as pl
from jax.experimental.pallas import tpu as pltpu
from jax.experimental.pallas import tpu_sc as plsc

_N_TAB = 49
_D = 128
_C = 128
_NBUF = 4
_PREF = 2
_NC = 2
_NS = 16
_NW = _NC * _NS


def _sc_body(nsteps, seq_hbm, idx_hbm, tab_hbm, out_hbm,
             tab_sh, idx_v, seq_v,
             isems, ssems, gsems, osems):
    cid = lax.axis_index("c")
    sid = lax.axis_index("s")
    wid = sid * _NC + cid
    rows_per_w = nsteps * _C
    w_base = wid * rows_per_w

    @pl.when(sid == 0)
    def _():
        pltpu.sync_copy(tab_hbm, tab_sh)
    plsc.subcore_barrier()

    def issue_in(g, b):
        base = w_base + g * _C
        pltpu.async_copy(idx_hbm.at[pl.ds(base, _C)], idx_v.at[b], isems.at[b])
        pltpu.async_copy(seq_hbm.at[pl.ds(base, _C)], seq_v.at[b], ssems.at[b])

    def wait_in(b):
        pltpu.make_async_copy(idx_hbm.at[pl.ds(0, _C)], idx_v.at[b],
                              isems.at[b]).wait()
        pltpu.make_async_copy(seq_hbm.at[pl.ds(0, _C)], seq_v.at[b],
                              ssems.at[b]).wait()

    for b in range(_PREF):
        issue_in(b, b)

    def group(grp, carry):
        for b in range(_NBUF):
            g = grp * _NBUF + b
            wait_in(b)
            for i in range(_C // 16):
                v = idx_v[b, pl.ds(i * 16, 16)]
                idx_v[b, pl.ds(i * 16, 16)] = jnp.where(v == -1, _N_TAB - 1, v)
            gather = pltpu.async_copy(tab_sh.at[idx_v.at[b]], seq_v.at[b],
                                      gsems.at[b], add=True)
            nxt = g + _PREF
            nb = (b + _PREF) % _NBUF

            @pl.when(nxt < nsteps)
            def _():
                @pl.when(g >= _NBUF - _PREF)
                def _():
                    pltpu.make_async_copy(
                        seq_v.at[nb], out_hbm.at[pl.ds(0, _C)],
                        osems.at[nb]).wait()
                issue_in(nxt, nb)

            gather.wait()
            pltpu.async_copy(seq_v.at[b], out_hbm.at[pl.ds(w_base + g * _C, _C)],
                             osems.at[b])
        return carry

    lax.fori_loop(0, nsteps // _NBUF, group, 0)

    for b in range(_NBUF):
        pltpu.make_async_copy(seq_v.at[b], out_hbm.at[pl.ds(0, _C)],
                              osems.at[b]).wait()


def kernel(seq, time_stamp, time_embed):
    B, L, D = seq.shape
    n = B * L
    seq2 = seq.reshape(n, D)
    idx = time_stamp.reshape(-1).astype(jnp.int32)
    nsteps = n // (_NW * _C)
    mesh = plsc.VectorSubcoreMesh(core_axis_name="c", subcore_axis_name="s")
    out = pl.kernel(
        functools.partial(_sc_body, nsteps),
        out_type=jax.ShapeDtypeStruct((n, D), jnp.float32),
        mesh=mesh,
        scratch_types=[
            pltpu.VMEM_SHARED((_N_TAB, _D), jnp.float32),
            pltpu.VMEM((_NBUF, _C), jnp.int32),
            pltpu.VMEM((_NBUF, _C, _D), jnp.float32),
            pltpu.SemaphoreType.DMA((_NBUF,)),
            pltpu.SemaphoreType.DMA((_NBUF,)),
            pltpu.SemaphoreType.DMA((_NBUF,)),
            pltpu.SemaphoreType.DMA((_NBUF,)),
        ],
    )(seq2, idx, time_embed)
    return out.reshape(B, L, D)
